# baseline (speedup 1.0000x reference)
import functools

import jax
import jax.numpy as jnp
from jax.experimental import pallas as pl

QK_DIM = 32
TOPK = 4
SCALE = QK_DIM ** (-0.5)


def _routing_kernel(q_ref, k_ref, w_ref, i_ref, *, n_keys):
    q = q_ref[0]
    k = k_ref[0]
    x = jax.lax.dot_general(
        q * SCALE, k,
        dimension_numbers=(((1,), (1,)), ((), ())),
        preferred_element_type=jnp.float32,
    )

    rb = x.shape[0]
    colf = jax.lax.broadcasted_iota(jnp.int32, (rb, n_keys), 1).astype(
        jnp.float32)
    big = float(n_keys)

    vals = []
    idxs = []
    for j in range(TOPK):
        m = jnp.max(x, axis=1, keepdims=True)
        cand = jnp.where(x == m, colf, big)
        a = jnp.min(cand, axis=1, keepdims=True)
        vals.append(m)
        idxs.append(a)
        if j < TOPK - 1:
            x = jnp.where(cand == a, -jnp.inf, x)

    topv = jnp.concatenate(vals, axis=1)
    topi = jnp.concatenate(idxs, axis=1).astype(jnp.int32)

    e = jnp.exp(topv - vals[0])
    w = e / jnp.sum(e, axis=1, keepdims=True)

    w_ref[0] = w
    i_ref[0] = topi


def kernel(query, key):
    b, n, d = query.shape
    rb = 1024
    grid = (b, n // rb)
    f = functools.partial(_routing_kernel, n_keys=n)
    w, i = pl.pallas_call(
        f,
        grid=grid,
        in_specs=[
            pl.BlockSpec((1, rb, d), lambda bi, ri: (bi, ri, 0)),
            pl.BlockSpec((1, n, d), lambda bi, ri: (bi, 0, 0)),
        ],
        out_specs=[
            pl.BlockSpec((1, rb, TOPK), lambda bi, ri: (bi, ri, 0)),
            pl.BlockSpec((1, rb, TOPK), lambda bi, ri: (bi, ri, 0)),
        ],
        out_shape=[
            jax.ShapeDtypeStruct((b, n, TOPK), jnp.float32),
            jax.ShapeDtypeStruct((b, n, TOPK), jnp.int32),
        ],
    )(query, key)
    return (w, i)


# speedup vs baseline: 1.0286x; 1.0286x over previous
import functools

import jax
import jax.numpy as jnp
from jax.experimental import pallas as pl

QK_DIM = 32
TOPK = 4
SCALE = QK_DIM ** (-0.5)
LW = 128


def _routing_kernel(q_ref, k_ref, w_ref, i_ref, *, n_keys):
    q = q_ref[0]
    k = k_ref[0]
    x = jax.lax.dot_general(
        q * SCALE, k,
        dimension_numbers=(((1,), (1,)), ((), ())),
        preferred_element_type=jnp.float32,
    )

    rb = x.shape[0]
    ns = n_keys // LW
    lane = jax.lax.broadcasted_iota(jnp.int32, (rb, LW), 1).astype(
        jnp.float32)
    xs = [x[:, c * LW:(c + 1) * LW] for c in range(ns)]
    cols = [lane + float(c * LW) for c in range(ns)]

    big = float(n_keys)
    vals = []
    idxs = []
    for j in range(TOPK):
        acc = xs[0]
        aid = cols[0]
        for c in range(1, ns):
            b = acc >= xs[c]
            acc = jnp.where(b, acc, xs[c])
            aid = jnp.where(b, aid, cols[c])
        m = jnp.max(acc, axis=1, keepdims=True)

        cand = jnp.where(acc == m, aid, big)
        a = jnp.min(cand, axis=1, keepdims=True)

        vals.append(m)
        idxs.append(a)
        if j < TOPK - 1:
            xs = [jnp.where(cols[c] == a, -jnp.inf, xs[c])
                  for c in range(ns)]

    topv = jnp.concatenate(vals, axis=1)
    topi = jnp.concatenate(idxs, axis=1).astype(jnp.int32)

    e = jnp.exp(topv - vals[0])
    w = e / jnp.sum(e, axis=1, keepdims=True)

    w_ref[0] = w
    i_ref[0] = topi


def kernel(query, key):
    b, n, d = query.shape
    rb = 1024
    grid = (b, n // rb)
    f = functools.partial(_routing_kernel, n_keys=n)
    w, i = pl.pallas_call(
        f,
        grid=grid,
        in_specs=[
            pl.BlockSpec((1, rb, d), lambda bi, ri: (bi, ri, 0)),
            pl.BlockSpec((1, n, d), lambda bi, ri: (bi, 0, 0)),
        ],
        out_specs=[
            pl.BlockSpec((1, rb, TOPK), lambda bi, ri: (bi, ri, 0)),
            pl.BlockSpec((1, rb, TOPK), lambda bi, ri: (bi, ri, 0)),
        ],
        out_shape=[
            jax.ShapeDtypeStruct((b, n, TOPK), jnp.float32),
            jax.ShapeDtypeStruct((b, n, TOPK), jnp.int32),
        ],
    )(query, key)
    return (w, i)
